# baseline (speedup 1.0000x reference)
import functools

import jax
import jax.numpy as jnp
from jax import lax
from jax.experimental import pallas as pl
from jax.experimental.pallas import tpu as pltpu
from jax.experimental.pallas import tpu_sc as plsc

B = 2
N = 10000
E = 160000
CIN = 128
COUT = 128

NPAD = 10240
NB = 1024
NTILE = 16
EPT = E // NTILE
CH = 80
NCHUNK = EPT // CH
ACCN = 10112
RPT = ACCN // NTILE


def _mlp_body(f_ref, w1_ref, b1_ref, w2_ref, b2_ref, o_ref):
    x = f_ref[0]
    h = jax.nn.relu(
        lax.dot_general(x, w1_ref[...], (((0,), (1,)), ((), ())),
                        preferred_element_type=jnp.float32)
        + b1_ref[...])
    y = (lax.dot_general(h, w2_ref[...], (((1,), (1,)), ((), ())),
                         preferred_element_type=jnp.float32)
         + b2_ref[...])
    o_ref[0] = y


def _mlp_rows(f_pad, W1, b1, W2, b2):
    grid = (B, NPAD // NB)
    return pl.pallas_call(
        _mlp_body,
        grid=grid,
        in_specs=[
            pl.BlockSpec((1, CIN, NB), lambda b, n: (b, 0, n)),
            pl.BlockSpec((COUT, CIN), lambda b, n: (0, 0)),
            pl.BlockSpec((1, COUT), lambda b, n: (0, 0)),
            pl.BlockSpec((COUT, COUT), lambda b, n: (0, 0)),
            pl.BlockSpec((1, COUT), lambda b, n: (0, 0)),
        ],
        out_specs=pl.BlockSpec((1, NB, COUT), lambda b, n: (b, n, 0)),
        out_shape=jax.ShapeDtypeStruct((B, NPAD, COUT), jnp.float32),
    )(f_pad, W1, b1.reshape(1, COUT), W2, b2.reshape(1, COUT))


def _sc_body(f2_hbm, nx_hbm, ny_hbm, idx_hbm, coef_hbm, out_hbm,
             idx_v, nx_v, ny_v, rows_v, w_v, coef_s, acc,
             sem_i, sem_g, sem_s):
    b = lax.axis_index("c")
    tid = lax.axis_index("s")

    pltpu.sync_copy(nx_hbm.at[b], nx_v)
    pltpu.sync_copy(ny_hbm.at[b], ny_v)
    pltpu.sync_copy(coef_hbm, coef_s)

    zi = jnp.zeros((16,), jnp.int32)
    w00 = plsc.load_gather(coef_s, [zi])
    w01 = plsc.load_gather(coef_s, [zi + 1])
    w10 = plsc.load_gather(coef_s, [zi + 2])
    w11 = plsc.load_gather(coef_s, [zi + 3])
    bs0 = plsc.load_gather(coef_s, [zi + 4])
    bs1 = plsc.load_gather(coef_s, [zi + 5])
    cc = plsc.load_gather(coef_s, [zi + 6])

    zv = jnp.zeros((16,), jnp.float32)

    def zero_body(r, _):
        for c in range(COUT // 16):
            rows_v[0, r, pl.ds(c * 16, 16)] = zv
        return 0

    lax.fori_loop(0, CH, zero_body, 0)
    for q in range(RPT // CH):
        pltpu.sync_copy(rows_v.at[0], acc.at[pl.ds(tid * RPT + q * CH, CH)])
    rem = RPT - (RPT // CH) * CH
    if rem:
        pltpu.sync_copy(
            rows_v.at[0, pl.ds(0, rem)],
            acc.at[pl.ds(tid * RPT + (RPT // CH) * CH, rem)])
    plsc.subcore_barrier()

    base = b * NPAD
    HC = CH // 2

    iota16 = lax.iota(jnp.int32, 16)
    rowv = []
    colv = []
    for g in range(CH // 16):
        ev = iota16 + g * 16
        rowv.append(ev // HC)
        colv.append(ev - (ev // HC) * HC)

    def issue_idx(j, s):
        pltpu.async_copy(idx_hbm.at[b, tid, j], idx_v.at[s], sem_i.at[s])

    def wait_idx(j, s):
        pltpu.make_async_copy(idx_hbm.at[b, tid, j], idx_v.at[s],
                              sem_i.at[s]).wait()

    def issue_gather(s, p):
        for r in range(2):
            pltpu.async_copy(f2_hbm.at[idx_v.at[s, r]],
                             rows_v.at[p, pl.ds(r * HC, HC)], sem_g.at[p])

    def wait_gather(s, p):
        for r in range(2):
            pltpu.make_async_copy(f2_hbm.at[idx_v.at[s, r]],
                                  rows_v.at[p, pl.ds(r * HC, HC)],
                                  sem_g.at[p]).wait()

    def issue_scatter(s, p, r):
        pltpu.async_copy(rows_v.at[p, pl.ds(r * HC, HC)],
                         acc.at[idx_v.at[s, 2 + r]],
                         sem_s.at[p * 2 + r], add=True)

    def wait_scatter(s, p, r):
        pltpu.make_async_copy(rows_v.at[p, pl.ds(r * HC, HC)],
                              acc.at[idx_v.at[s, 2 + r]],
                              sem_s.at[p * 2 + r]).wait()

    def compute_w(s):
        sv_s = jnp.zeros((16,), jnp.int32) + s
        for g in range(CH // 16):
            sv = plsc.load_gather(idx_v, [sv_s, rowv[g], colv[g]]) - base
            tv = plsc.load_gather(idx_v, [sv_s, rowv[g] + 2, colv[g]])
            xs = plsc.load_gather(nx_v, [sv])
            ys = plsc.load_gather(ny_v, [sv])
            xt = plsc.load_gather(nx_v, [tv])
            yt = plsc.load_gather(ny_v, [tv])
            e0 = xs - xt
            e1 = ys - yt
            s0 = w00 * e0 + w01 * e1 + bs0
            s1 = w10 * e0 + w11 * e1 + bs1
            d = e0 * s0 + e1 * s1
            w_v[pl.ds(g * 16, 16)] = cc * jnp.exp(d)

    def scale_half(p, r):
        def scale_body(e4, _):
            e = r * HC + e4 * 4
            for u in range(4):
                wb = plsc.load_gather(
                    w_v, [jnp.zeros((16,), jnp.int32) + (e + u)])
                for c in range(COUT // 16):
                    rows_v[p, e + u, pl.ds(c * 16, 16)] = (
                        rows_v[p, e + u, pl.ds(c * 16, 16)] * wb)
            return 0

        lax.fori_loop(0, HC // 4, scale_body, 0)

    def process_chunk(s, p):
        compute_w(s)
        scale_half(p, 0)
        issue_scatter(s, p, 0)
        scale_half(p, 1)
        issue_scatter(s, p, 1)

    issue_idx(0, 0)
    issue_idx(1, 1)
    issue_idx(2, 2)
    wait_idx(0, 0)
    issue_gather(0, 0)

    def quad_body(i, _):
        j0 = i * 4
        for k in range(4):
            j = j0 + k
            s = k
            p = k & 1
            sn = (k + 1) & 3
            pn = p ^ 1
            wait_idx(j + 1, sn)
            if k == 0:
                @pl.when(i >= 1)
                def _():
                    wait_scatter(3, pn, 0)
                    wait_scatter(3, pn, 1)
            else:
                wait_scatter(k - 1, pn, 0)
                wait_scatter(k - 1, pn, 1)
            issue_gather(sn, pn)
            wait_gather(s, p)
            process_chunk(s, p)
            if k < 2:
                issue_idx(j + 3, (k + 3) & 3)
            else:
                @pl.when(i <= 29)
                def _():
                    issue_idx(j + 3, (k + 3) & 3)
        return 0

    lax.fori_loop(0, NCHUNK // 4, quad_body, 0)

    wait_scatter(3, 1, 0)
    wait_scatter(3, 1, 1)
    wait_gather(0, 0)
    process_chunk(0, 0)
    wait_scatter(0, 0, 0)
    wait_scatter(0, 0, 1)

    plsc.subcore_barrier()

    pltpu.sync_copy(acc.at[pl.ds(tid * RPT, RPT)],
                    out_hbm.at[pl.ds(b * ACCN + tid * RPT, RPT)])


def _sc_scatter(f2_flat, nx, ny, idx_r, coef):
    mesh = plsc.VectorSubcoreMesh(core_axis_name="c", subcore_axis_name="s")
    kern = functools.partial(
        pl.kernel, mesh=mesh,
        out_type=jax.ShapeDtypeStruct((B * ACCN, COUT), jnp.float32),
        scratch_types=[
            pltpu.VMEM((4, 4, CH // 2), jnp.int32),
            pltpu.VMEM((N,), jnp.float32),
            pltpu.VMEM((N,), jnp.float32),
            pltpu.VMEM((2, CH, COUT), jnp.float32),
            pltpu.VMEM((CH,), jnp.float32),
            pltpu.VMEM((16,), jnp.float32),
            pltpu.VMEM_SHARED((ACCN, COUT), jnp.float32),
            pltpu.SemaphoreType.DMA((4,)),
            pltpu.SemaphoreType.DMA((2,)),
            pltpu.SemaphoreType.DMA((4,)),
        ],
        compiler_params=pltpu.CompilerParams(needs_layout_passes=False),
    )(_sc_body)
    return kern(f2_flat, nx, ny, idx_r, coef)


def kernel(f, nodes, edges_index, W1, b1, W2, b2, Wsig, bsig, Cparam):
    f_pad = jnp.pad(f, ((0, 0), (0, 0), (0, NPAD - N)))
    f2 = _mlp_rows(f_pad, W1, b1, W2, b2)
    f2_flat = f2.reshape(B * NPAD, COUT)

    src = edges_index[..., 0]
    tgt = edges_index[..., 1]
    src_off = src + (jnp.arange(B, dtype=jnp.int32) * NPAD)[:, None]
    idx_r = jnp.stack(
        [src_off.reshape(B, NTILE, NCHUNK, CH),
         tgt.reshape(B, NTILE, NCHUNK, CH)],
        axis=3).reshape(B, NTILE, NCHUNK, 4, CH // 2)

    nx = nodes[..., 0]
    ny = nodes[..., 1]

    coef = jnp.concatenate([
        Wsig.reshape(4), bsig.reshape(2), Cparam.reshape(1),
        jnp.zeros((9,), jnp.float32)])

    out_flat = _sc_scatter(f2_flat, nx, ny, idx_r, coef)
    out = out_flat.reshape(B, ACCN, COUT)[:, :N]
    return jnp.transpose(out, (0, 2, 1))


# speedup vs baseline: 1.0268x; 1.0268x over previous
import functools

import jax
import jax.numpy as jnp
from jax import lax
from jax.experimental import pallas as pl
from jax.experimental.pallas import tpu as pltpu
from jax.experimental.pallas import tpu_sc as plsc

B = 2
N = 10000
E = 160000
CIN = 128
COUT = 128

NPAD = 10240
NB = 2048
NTILE = 16
EPT = E // NTILE
CH = 80
NCHUNK = EPT // CH
ACCN = 10112
RPT = ACCN // NTILE


def _mlp_body(f_ref, w1_ref, b1_ref, w2_ref, b2_ref, o_ref):
    x = f_ref[0]
    h = jax.nn.relu(
        lax.dot_general(x, w1_ref[...], (((0,), (1,)), ((), ())),
                        preferred_element_type=jnp.float32)
        + b1_ref[...])
    y = (lax.dot_general(h, w2_ref[...], (((1,), (1,)), ((), ())),
                         preferred_element_type=jnp.float32)
         + b2_ref[...])
    o_ref[0] = y


def _mlp_rows(f_pad, W1, b1, W2, b2):
    grid = (B, NPAD // NB)
    return pl.pallas_call(
        _mlp_body,
        grid=grid,
        in_specs=[
            pl.BlockSpec((1, CIN, NB), lambda b, n: (b, 0, n)),
            pl.BlockSpec((COUT, CIN), lambda b, n: (0, 0)),
            pl.BlockSpec((1, COUT), lambda b, n: (0, 0)),
            pl.BlockSpec((COUT, COUT), lambda b, n: (0, 0)),
            pl.BlockSpec((1, COUT), lambda b, n: (0, 0)),
        ],
        out_specs=pl.BlockSpec((1, NB, COUT), lambda b, n: (b, n, 0)),
        out_shape=jax.ShapeDtypeStruct((B, NPAD, COUT), jnp.float32),
    )(f_pad, W1, b1.reshape(1, COUT), W2, b2.reshape(1, COUT))


def _sc_body(f2_hbm, nx_hbm, ny_hbm, idx_hbm, coef_hbm, out_hbm,
             idx_v, nx_v, ny_v, rows_v, w_v, coef_s, acc,
             sem_i, sem_g, sem_s):
    b = lax.axis_index("c")
    tid = lax.axis_index("s")

    pltpu.sync_copy(nx_hbm.at[b], nx_v)
    pltpu.sync_copy(ny_hbm.at[b], ny_v)
    pltpu.sync_copy(coef_hbm, coef_s)

    zi = jnp.zeros((16,), jnp.int32)
    w00 = plsc.load_gather(coef_s, [zi])
    w01 = plsc.load_gather(coef_s, [zi + 1])
    w10 = plsc.load_gather(coef_s, [zi + 2])
    w11 = plsc.load_gather(coef_s, [zi + 3])
    bs0 = plsc.load_gather(coef_s, [zi + 4])
    bs1 = plsc.load_gather(coef_s, [zi + 5])
    cc = plsc.load_gather(coef_s, [zi + 6])

    zv = jnp.zeros((16,), jnp.float32)

    def zero_body(r, _):
        for c in range(COUT // 16):
            rows_v[0, r, pl.ds(c * 16, 16)] = zv
        return 0

    lax.fori_loop(0, CH, zero_body, 0)
    for q in range(RPT // CH):
        pltpu.sync_copy(rows_v.at[0], acc.at[pl.ds(tid * RPT + q * CH, CH)])
    rem = RPT - (RPT // CH) * CH
    if rem:
        pltpu.sync_copy(
            rows_v.at[0, pl.ds(0, rem)],
            acc.at[pl.ds(tid * RPT + (RPT // CH) * CH, rem)])
    plsc.subcore_barrier()

    base = b * NPAD
    HC = CH // 2

    iota16 = lax.iota(jnp.int32, 16)
    rowv = []
    colv = []
    for g in range(CH // 16):
        ev = iota16 + g * 16
        rowv.append(ev // HC)
        colv.append(ev - (ev // HC) * HC)

    def issue_idx(j, s):
        pltpu.async_copy(idx_hbm.at[b, tid, j], idx_v.at[s], sem_i.at[s])

    def wait_idx(j, s):
        pltpu.make_async_copy(idx_hbm.at[b, tid, j], idx_v.at[s],
                              sem_i.at[s]).wait()

    def issue_gather(s, p):
        for r in range(2):
            pltpu.async_copy(f2_hbm.at[idx_v.at[s, r]],
                             rows_v.at[p, pl.ds(r * HC, HC)], sem_g.at[p])

    def wait_gather(s, p):
        for r in range(2):
            pltpu.make_async_copy(f2_hbm.at[idx_v.at[s, r]],
                                  rows_v.at[p, pl.ds(r * HC, HC)],
                                  sem_g.at[p]).wait()

    def issue_scatter(s, p, r):
        pltpu.async_copy(rows_v.at[p, pl.ds(r * HC, HC)],
                         acc.at[idx_v.at[s, 2 + r]],
                         sem_s.at[p * 2 + r], add=True)

    def wait_scatter(s, p, r):
        pltpu.make_async_copy(rows_v.at[p, pl.ds(r * HC, HC)],
                              acc.at[idx_v.at[s, 2 + r]],
                              sem_s.at[p * 2 + r]).wait()

    def compute_w(s):
        sv_s = jnp.zeros((16,), jnp.int32) + s
        for g in range(CH // 16):
            sv = plsc.load_gather(idx_v, [sv_s, rowv[g], colv[g]]) - base
            tv = plsc.load_gather(idx_v, [sv_s, rowv[g] + 2, colv[g]])
            xs = plsc.load_gather(nx_v, [sv])
            ys = plsc.load_gather(ny_v, [sv])
            xt = plsc.load_gather(nx_v, [tv])
            yt = plsc.load_gather(ny_v, [tv])
            e0 = xs - xt
            e1 = ys - yt
            s0 = w00 * e0 + w01 * e1 + bs0
            s1 = w10 * e0 + w11 * e1 + bs1
            d = e0 * s0 + e1 * s1
            w_v[pl.ds(g * 16, 16)] = cc * jnp.exp(d)

    def scale_half(p, r):
        def scale_body(e4, _):
            e = r * HC + e4 * 4
            for u in range(4):
                wb = plsc.load_gather(
                    w_v, [jnp.zeros((16,), jnp.int32) + (e + u)])
                for c in range(COUT // 16):
                    rows_v[p, e + u, pl.ds(c * 16, 16)] = (
                        rows_v[p, e + u, pl.ds(c * 16, 16)] * wb)
            return 0

        lax.fori_loop(0, HC // 4, scale_body, 0)

    def process_chunk(s, p):
        compute_w(s)
        scale_half(p, 0)
        issue_scatter(s, p, 0)
        scale_half(p, 1)
        issue_scatter(s, p, 1)

    issue_idx(0, 0)
    issue_idx(1, 1)
    issue_idx(2, 2)
    wait_idx(0, 0)
    issue_gather(0, 0)

    def quad_body(i, _):
        j0 = i * 4
        for k in range(4):
            j = j0 + k
            s = k
            p = k & 1
            sn = (k + 1) & 3
            pn = p ^ 1
            wait_idx(j + 1, sn)
            if k == 0:
                @pl.when(i >= 1)
                def _():
                    wait_scatter(3, pn, 0)
                    wait_scatter(3, pn, 1)
            else:
                wait_scatter(k - 1, pn, 0)
                wait_scatter(k - 1, pn, 1)
            issue_gather(sn, pn)
            wait_gather(s, p)
            process_chunk(s, p)
            if k < 2:
                issue_idx(j + 3, (k + 3) & 3)
            else:
                @pl.when(i <= 29)
                def _():
                    issue_idx(j + 3, (k + 3) & 3)
        return 0

    lax.fori_loop(0, NCHUNK // 4, quad_body, 0)

    wait_scatter(3, 1, 0)
    wait_scatter(3, 1, 1)
    wait_gather(0, 0)
    process_chunk(0, 0)
    wait_scatter(0, 0, 0)
    wait_scatter(0, 0, 1)

    plsc.subcore_barrier()

    pltpu.sync_copy(acc.at[pl.ds(tid * RPT, RPT)],
                    out_hbm.at[pl.ds(b * ACCN + tid * RPT, RPT)])


def _sc_scatter(f2_flat, nx, ny, idx_r, coef):
    mesh = plsc.VectorSubcoreMesh(core_axis_name="c", subcore_axis_name="s")
    kern = functools.partial(
        pl.kernel, mesh=mesh,
        out_type=jax.ShapeDtypeStruct((B * ACCN, COUT), jnp.float32),
        scratch_types=[
            pltpu.VMEM((4, 4, CH // 2), jnp.int32),
            pltpu.VMEM((N,), jnp.float32),
            pltpu.VMEM((N,), jnp.float32),
            pltpu.VMEM((2, CH, COUT), jnp.float32),
            pltpu.VMEM((CH,), jnp.float32),
            pltpu.VMEM((16,), jnp.float32),
            pltpu.VMEM_SHARED((ACCN, COUT), jnp.float32),
            pltpu.SemaphoreType.DMA((4,)),
            pltpu.SemaphoreType.DMA((2,)),
            pltpu.SemaphoreType.DMA((4,)),
        ],
        compiler_params=pltpu.CompilerParams(needs_layout_passes=False),
    )(_sc_body)
    return kern(f2_flat, nx, ny, idx_r, coef)


def kernel(f, nodes, edges_index, W1, b1, W2, b2, Wsig, bsig, Cparam):
    f_pad = jnp.pad(f, ((0, 0), (0, 0), (0, NPAD - N)))
    f2 = _mlp_rows(f_pad, W1, b1, W2, b2)
    f2_flat = f2.reshape(B * NPAD, COUT)

    src = edges_index[..., 0]
    tgt = edges_index[..., 1]
    src_off = src + (jnp.arange(B, dtype=jnp.int32) * NPAD)[:, None]
    idx_r = jnp.stack(
        [src_off.reshape(B, NTILE, NCHUNK, CH),
         tgt.reshape(B, NTILE, NCHUNK, CH)],
        axis=3).reshape(B, NTILE, NCHUNK, 4, CH // 2)

    nx = nodes[..., 0]
    ny = nodes[..., 1]

    coef = jnp.concatenate([
        Wsig.reshape(4), bsig.reshape(2), Cparam.reshape(1),
        jnp.zeros((9,), jnp.float32)])

    out_flat = _sc_scatter(f2_flat, nx, ny, idx_r, coef)
    out = out_flat.reshape(B, ACCN, COUT)[:, :N]
    return jnp.transpose(out, (0, 2, 1))


# speedup vs baseline: 1.0373x; 1.0103x over previous
import functools

import jax
import jax.numpy as jnp
from jax import lax
from jax.experimental import pallas as pl
from jax.experimental.pallas import tpu as pltpu
from jax.experimental.pallas import tpu_sc as plsc

B = 2
N = 10000
E = 160000
CIN = 128
COUT = 128

NPAD = 10240
NB = 5120
NTILE = 16
EPT = E // NTILE
CH = 80
NCHUNK = EPT // CH
ACCN = 10112
RPT = ACCN // NTILE


def _mlp_body(f_ref, w1_ref, b1_ref, w2_ref, b2_ref, o_ref):
    x = f_ref[0]
    h = jax.nn.relu(
        lax.dot_general(x, w1_ref[...], (((0,), (1,)), ((), ())),
                        preferred_element_type=jnp.float32)
        + b1_ref[...])
    y = (lax.dot_general(h, w2_ref[...], (((1,), (1,)), ((), ())),
                         preferred_element_type=jnp.float32)
         + b2_ref[...])
    o_ref[0] = y


def _mlp_rows(f_pad, W1, b1, W2, b2):
    grid = (B, NPAD // NB)
    return pl.pallas_call(
        _mlp_body,
        grid=grid,
        in_specs=[
            pl.BlockSpec((1, CIN, NB), lambda b, n: (b, 0, n)),
            pl.BlockSpec((COUT, CIN), lambda b, n: (0, 0)),
            pl.BlockSpec((1, COUT), lambda b, n: (0, 0)),
            pl.BlockSpec((COUT, COUT), lambda b, n: (0, 0)),
            pl.BlockSpec((1, COUT), lambda b, n: (0, 0)),
        ],
        out_specs=pl.BlockSpec((1, NB, COUT), lambda b, n: (b, n, 0)),
        out_shape=jax.ShapeDtypeStruct((B, NPAD, COUT), jnp.float32),
    )(f_pad, W1, b1.reshape(1, COUT), W2, b2.reshape(1, COUT))


def _sc_body(f2_hbm, nx_hbm, ny_hbm, idx_hbm, coef_hbm, out_hbm,
             idx_v, nx_v, ny_v, rows_v, w_v, coef_s, acc,
             sem_i, sem_g, sem_s):
    b = lax.axis_index("c")
    tid = lax.axis_index("s")

    pltpu.sync_copy(nx_hbm.at[b], nx_v)
    pltpu.sync_copy(ny_hbm.at[b], ny_v)
    pltpu.sync_copy(coef_hbm, coef_s)

    zi = jnp.zeros((16,), jnp.int32)
    w00 = plsc.load_gather(coef_s, [zi])
    w01 = plsc.load_gather(coef_s, [zi + 1])
    w10 = plsc.load_gather(coef_s, [zi + 2])
    w11 = plsc.load_gather(coef_s, [zi + 3])
    bs0 = plsc.load_gather(coef_s, [zi + 4])
    bs1 = plsc.load_gather(coef_s, [zi + 5])
    cc = plsc.load_gather(coef_s, [zi + 6])

    zv = jnp.zeros((16,), jnp.float32)

    def zero_body(r, _):
        for c in range(COUT // 16):
            rows_v[0, r, pl.ds(c * 16, 16)] = zv
        return 0

    lax.fori_loop(0, CH, zero_body, 0)
    for q in range(RPT // CH):
        pltpu.sync_copy(rows_v.at[0], acc.at[pl.ds(tid * RPT + q * CH, CH)])
    rem = RPT - (RPT // CH) * CH
    if rem:
        pltpu.sync_copy(
            rows_v.at[0, pl.ds(0, rem)],
            acc.at[pl.ds(tid * RPT + (RPT // CH) * CH, rem)])
    plsc.subcore_barrier()

    base = b * NPAD
    HC = CH // 2

    iota16 = lax.iota(jnp.int32, 16)
    rowv = []
    colv = []
    for g in range(CH // 16):
        ev = iota16 + g * 16
        rowv.append(ev // HC)
        colv.append(ev - (ev // HC) * HC)

    def issue_idx(j, s):
        pltpu.async_copy(idx_hbm.at[b, tid, j], idx_v.at[s], sem_i.at[s])

    def wait_idx(j, s):
        pltpu.make_async_copy(idx_hbm.at[b, tid, j], idx_v.at[s],
                              sem_i.at[s]).wait()

    def issue_gather(s, p):
        for r in range(2):
            pltpu.async_copy(f2_hbm.at[idx_v.at[s, r]],
                             rows_v.at[p, pl.ds(r * HC, HC)], sem_g.at[p])

    def wait_gather(s, p):
        for r in range(2):
            pltpu.make_async_copy(f2_hbm.at[idx_v.at[s, r]],
                                  rows_v.at[p, pl.ds(r * HC, HC)],
                                  sem_g.at[p]).wait()

    def issue_scatter(s, p, r):
        pltpu.async_copy(rows_v.at[p, pl.ds(r * HC, HC)],
                         acc.at[idx_v.at[s, 2 + r]],
                         sem_s.at[p * 2 + r], add=True)

    def wait_scatter(s, p, r):
        pltpu.make_async_copy(rows_v.at[p, pl.ds(r * HC, HC)],
                              acc.at[idx_v.at[s, 2 + r]],
                              sem_s.at[p * 2 + r]).wait()

    def compute_w(s):
        sv_s = jnp.zeros((16,), jnp.int32) + s
        for g in range(CH // 16):
            sv = plsc.load_gather(idx_v, [sv_s, rowv[g], colv[g]]) - base
            tv = plsc.load_gather(idx_v, [sv_s, rowv[g] + 2, colv[g]])
            xs = plsc.load_gather(nx_v, [sv])
            ys = plsc.load_gather(ny_v, [sv])
            xt = plsc.load_gather(nx_v, [tv])
            yt = plsc.load_gather(ny_v, [tv])
            e0 = xs - xt
            e1 = ys - yt
            s0 = w00 * e0 + w01 * e1 + bs0
            s1 = w10 * e0 + w11 * e1 + bs1
            d = e0 * s0 + e1 * s1
            w_v[pl.ds(g * 16, 16)] = cc * jnp.exp(d)

    def scale_half(p, r):
        def scale_body(e4, _):
            e = r * HC + e4 * 4
            for u in range(4):
                wb = plsc.load_gather(
                    w_v, [jnp.zeros((16,), jnp.int32) + (e + u)])
                for c in range(COUT // 16):
                    rows_v[p, e + u, pl.ds(c * 16, 16)] = (
                        rows_v[p, e + u, pl.ds(c * 16, 16)] * wb)
            return 0

        lax.fori_loop(0, HC // 4, scale_body, 0)

    def process_chunk(s, p):
        compute_w(s)
        scale_half(p, 0)
        issue_scatter(s, p, 0)
        scale_half(p, 1)
        issue_scatter(s, p, 1)

    issue_idx(0, 0)
    issue_idx(1, 1)
    issue_idx(2, 2)
    wait_idx(0, 0)
    issue_gather(0, 0)

    def quad_body(i, _):
        j0 = i * 4
        for k in range(4):
            j = j0 + k
            s = k
            p = k & 1
            sn = (k + 1) & 3
            pn = p ^ 1
            wait_idx(j + 1, sn)
            if k == 0:
                @pl.when(i >= 1)
                def _():
                    wait_scatter(3, pn, 0)
                    wait_scatter(3, pn, 1)
            else:
                wait_scatter(k - 1, pn, 0)
                wait_scatter(k - 1, pn, 1)
            issue_gather(sn, pn)
            wait_gather(s, p)
            process_chunk(s, p)
            if k < 2:
                issue_idx(j + 3, (k + 3) & 3)
            else:
                @pl.when(i <= 29)
                def _():
                    issue_idx(j + 3, (k + 3) & 3)
        return 0

    lax.fori_loop(0, NCHUNK // 4, quad_body, 0)

    wait_scatter(3, 1, 0)
    wait_scatter(3, 1, 1)
    wait_gather(0, 0)
    process_chunk(0, 0)
    wait_scatter(0, 0, 0)
    wait_scatter(0, 0, 1)

    plsc.subcore_barrier()

    pltpu.sync_copy(acc.at[pl.ds(tid * RPT, RPT)],
                    out_hbm.at[pl.ds(b * ACCN + tid * RPT, RPT)])


def _sc_scatter(f2_flat, nx, ny, idx_r, coef):
    mesh = plsc.VectorSubcoreMesh(core_axis_name="c", subcore_axis_name="s")
    kern = functools.partial(
        pl.kernel, mesh=mesh,
        out_type=jax.ShapeDtypeStruct((B * ACCN, COUT), jnp.float32),
        scratch_types=[
            pltpu.VMEM((4, 4, CH // 2), jnp.int32),
            pltpu.VMEM((N,), jnp.float32),
            pltpu.VMEM((N,), jnp.float32),
            pltpu.VMEM((2, CH, COUT), jnp.float32),
            pltpu.VMEM((CH,), jnp.float32),
            pltpu.VMEM((16,), jnp.float32),
            pltpu.VMEM_SHARED((ACCN, COUT), jnp.float32),
            pltpu.SemaphoreType.DMA((4,)),
            pltpu.SemaphoreType.DMA((2,)),
            pltpu.SemaphoreType.DMA((4,)),
        ],
        compiler_params=pltpu.CompilerParams(needs_layout_passes=False),
    )(_sc_body)
    return kern(f2_flat, nx, ny, idx_r, coef)


def kernel(f, nodes, edges_index, W1, b1, W2, b2, Wsig, bsig, Cparam):
    f_pad = jnp.pad(f, ((0, 0), (0, 0), (0, NPAD - N)))
    f2 = _mlp_rows(f_pad, W1, b1, W2, b2)
    f2_flat = f2.reshape(B * NPAD, COUT)

    src = edges_index[..., 0]
    tgt = edges_index[..., 1]
    src_off = src + (jnp.arange(B, dtype=jnp.int32) * NPAD)[:, None]
    idx_r = jnp.stack(
        [src_off.reshape(B, NTILE, NCHUNK, CH),
         tgt.reshape(B, NTILE, NCHUNK, CH)],
        axis=3).reshape(B, NTILE, NCHUNK, 4, CH // 2)

    nx = nodes[..., 0]
    ny = nodes[..., 1]

    coef = jnp.concatenate([
        Wsig.reshape(4), bsig.reshape(2), Cparam.reshape(1),
        jnp.zeros((9,), jnp.float32)])

    out_flat = _sc_scatter(f2_flat, nx, ny, idx_r, coef)
    out = out_flat.reshape(B, ACCN, COUT)[:, :N]
    return jnp.transpose(out, (0, 2, 1))


# speedup vs baseline: 1.0392x; 1.0018x over previous
import functools

import jax
import jax.numpy as jnp
from jax import lax
from jax.experimental import pallas as pl
from jax.experimental.pallas import tpu as pltpu
from jax.experimental.pallas import tpu_sc as plsc

B = 2
N = 10000
E = 160000
CIN = 128
COUT = 128

NPAD = 10240
NB = 5120
NTILE = 16
EPT = E // NTILE
CH = 80
NCHUNK = EPT // CH
ACCN = 10112
RPT = ACCN // NTILE


def _mlp_body(f_ref, w1_ref, b1_ref, w2_ref, b2_ref, o_ref):
    x = f_ref[0]
    h = jax.nn.relu(
        lax.dot_general(x, w1_ref[...], (((0,), (1,)), ((), ())),
                        preferred_element_type=jnp.float32)
        + b1_ref[...])
    y = (lax.dot_general(h, w2_ref[...], (((1,), (1,)), ((), ())),
                         preferred_element_type=jnp.float32)
         + b2_ref[...])
    o_ref[0] = y


def _mlp_rows(f_pad, W1, b1, W2, b2):
    grid = (B, NPAD // NB)
    return pl.pallas_call(
        _mlp_body,
        grid=grid,
        in_specs=[
            pl.BlockSpec((1, CIN, NB), lambda b, n: (b, 0, n)),
            pl.BlockSpec((COUT, CIN), lambda b, n: (0, 0)),
            pl.BlockSpec((1, COUT), lambda b, n: (0, 0)),
            pl.BlockSpec((COUT, COUT), lambda b, n: (0, 0)),
            pl.BlockSpec((1, COUT), lambda b, n: (0, 0)),
        ],
        out_specs=pl.BlockSpec((1, NB, COUT), lambda b, n: (b, n, 0)),
        out_shape=jax.ShapeDtypeStruct((B, NPAD, COUT), jnp.float32),
    )(f_pad, W1, b1.reshape(1, COUT), W2, b2.reshape(1, COUT))


def _sc_body(f2_hbm, nx_hbm, ny_hbm, idx_hbm, coef_hbm, out_hbm,
             idx_v, nx_v, ny_v, rows_v, w_v, coef_s, acc,
             sem_i, sem_g, sem_s):
    b = lax.axis_index("c")
    tid = lax.axis_index("s")

    for s0 in range(3):
        pltpu.async_copy(idx_hbm.at[b, tid, s0], idx_v.at[s0], sem_i.at[s0])

    pltpu.sync_copy(nx_hbm.at[b], nx_v)
    pltpu.sync_copy(ny_hbm.at[b], ny_v)
    pltpu.sync_copy(coef_hbm, coef_s)

    zi = jnp.zeros((16,), jnp.int32)
    w00 = plsc.load_gather(coef_s, [zi])
    w01 = plsc.load_gather(coef_s, [zi + 1])
    w10 = plsc.load_gather(coef_s, [zi + 2])
    w11 = plsc.load_gather(coef_s, [zi + 3])
    bs0 = plsc.load_gather(coef_s, [zi + 4])
    bs1 = plsc.load_gather(coef_s, [zi + 5])
    cc = plsc.load_gather(coef_s, [zi + 6])

    zv = jnp.zeros((16,), jnp.float32)

    def zero_body(r, _):
        for c in range(COUT // 16):
            rows_v[0, r, pl.ds(c * 16, 16)] = zv
        return 0

    lax.fori_loop(0, CH, zero_body, 0)
    rem = RPT - (RPT // CH) * CH
    for q in range(RPT // CH):
        pltpu.async_copy(rows_v.at[0], acc.at[pl.ds(tid * RPT + q * CH, CH)],
                         sem_g.at[0])
    pltpu.async_copy(rows_v.at[0, pl.ds(0, rem)],
                     acc.at[pl.ds(tid * RPT + (RPT // CH) * CH, rem)],
                     sem_g.at[0])
    for q in range(RPT // CH):
        pltpu.make_async_copy(
            rows_v.at[0], acc.at[pl.ds(tid * RPT + q * CH, CH)],
            sem_g.at[0]).wait()
    pltpu.make_async_copy(
        rows_v.at[0, pl.ds(0, rem)],
        acc.at[pl.ds(tid * RPT + (RPT // CH) * CH, rem)],
        sem_g.at[0]).wait()
    plsc.subcore_barrier()

    base = b * NPAD
    HC = CH // 2

    iota16 = lax.iota(jnp.int32, 16)
    rowv = []
    colv = []
    for g in range(CH // 16):
        ev = iota16 + g * 16
        rowv.append(ev // HC)
        colv.append(ev - (ev // HC) * HC)

    def issue_idx(j, s):
        pltpu.async_copy(idx_hbm.at[b, tid, j], idx_v.at[s], sem_i.at[s])

    def wait_idx(j, s):
        pltpu.make_async_copy(idx_hbm.at[b, tid, j], idx_v.at[s],
                              sem_i.at[s]).wait()

    def issue_gather(s, p):
        for r in range(2):
            pltpu.async_copy(f2_hbm.at[idx_v.at[s, r]],
                             rows_v.at[p, pl.ds(r * HC, HC)], sem_g.at[p])

    def wait_gather(s, p):
        for r in range(2):
            pltpu.make_async_copy(f2_hbm.at[idx_v.at[s, r]],
                                  rows_v.at[p, pl.ds(r * HC, HC)],
                                  sem_g.at[p]).wait()

    def issue_scatter(s, p, r):
        pltpu.async_copy(rows_v.at[p, pl.ds(r * HC, HC)],
                         acc.at[idx_v.at[s, 2 + r]],
                         sem_s.at[p * 2 + r], add=True)

    def wait_scatter(s, p, r):
        pltpu.make_async_copy(rows_v.at[p, pl.ds(r * HC, HC)],
                              acc.at[idx_v.at[s, 2 + r]],
                              sem_s.at[p * 2 + r]).wait()

    def compute_w(s):
        sv_s = jnp.zeros((16,), jnp.int32) + s
        for g in range(CH // 16):
            sv = plsc.load_gather(idx_v, [sv_s, rowv[g], colv[g]]) - base
            tv = plsc.load_gather(idx_v, [sv_s, rowv[g] + 2, colv[g]])
            xs = plsc.load_gather(nx_v, [sv])
            ys = plsc.load_gather(ny_v, [sv])
            xt = plsc.load_gather(nx_v, [tv])
            yt = plsc.load_gather(ny_v, [tv])
            e0 = xs - xt
            e1 = ys - yt
            s0 = w00 * e0 + w01 * e1 + bs0
            s1 = w10 * e0 + w11 * e1 + bs1
            d = e0 * s0 + e1 * s1
            w_v[pl.ds(g * 16, 16)] = cc * jnp.exp(d)

    def scale_half(p, r):
        def scale_body(e4, _):
            e = r * HC + e4 * 4
            for u in range(4):
                wb = plsc.load_gather(
                    w_v, [jnp.zeros((16,), jnp.int32) + (e + u)])
                for c in range(COUT // 16):
                    rows_v[p, e + u, pl.ds(c * 16, 16)] = (
                        rows_v[p, e + u, pl.ds(c * 16, 16)] * wb)
            return 0

        lax.fori_loop(0, HC // 4, scale_body, 0)

    def process_chunk(s, p):
        compute_w(s)
        scale_half(p, 0)
        issue_scatter(s, p, 0)
        scale_half(p, 1)
        issue_scatter(s, p, 1)

    wait_idx(0, 0)
    issue_gather(0, 0)

    def quad_body(i, _):
        j0 = i * 4
        for k in range(4):
            j = j0 + k
            s = k
            p = k & 1
            sn = (k + 1) & 3
            pn = p ^ 1
            wait_idx(j + 1, sn)
            if k == 0:
                @pl.when(i >= 1)
                def _():
                    wait_scatter(3, pn, 0)
                    wait_scatter(3, pn, 1)
            else:
                wait_scatter(k - 1, pn, 0)
                wait_scatter(k - 1, pn, 1)
            issue_gather(sn, pn)
            wait_gather(s, p)
            process_chunk(s, p)
            if k < 2:
                issue_idx(j + 3, (k + 3) & 3)
            else:
                @pl.when(i <= 29)
                def _():
                    issue_idx(j + 3, (k + 3) & 3)
        return 0

    lax.fori_loop(0, NCHUNK // 4, quad_body, 0)

    wait_scatter(3, 1, 0)
    wait_scatter(3, 1, 1)
    wait_gather(0, 0)
    process_chunk(0, 0)
    wait_scatter(0, 0, 0)
    wait_scatter(0, 0, 1)

    plsc.subcore_barrier()

    pltpu.sync_copy(acc.at[pl.ds(tid * RPT, RPT)],
                    out_hbm.at[pl.ds(b * ACCN + tid * RPT, RPT)])


def _sc_scatter(f2_flat, nx, ny, idx_r, coef):
    mesh = plsc.VectorSubcoreMesh(core_axis_name="c", subcore_axis_name="s")
    kern = functools.partial(
        pl.kernel, mesh=mesh,
        out_type=jax.ShapeDtypeStruct((B * ACCN, COUT), jnp.float32),
        scratch_types=[
            pltpu.VMEM((4, 4, CH // 2), jnp.int32),
            pltpu.VMEM((N,), jnp.float32),
            pltpu.VMEM((N,), jnp.float32),
            pltpu.VMEM((2, CH, COUT), jnp.float32),
            pltpu.VMEM((CH,), jnp.float32),
            pltpu.VMEM((16,), jnp.float32),
            pltpu.VMEM_SHARED((ACCN, COUT), jnp.float32),
            pltpu.SemaphoreType.DMA((4,)),
            pltpu.SemaphoreType.DMA((2,)),
            pltpu.SemaphoreType.DMA((4,)),
        ],
        compiler_params=pltpu.CompilerParams(needs_layout_passes=False),
    )(_sc_body)
    return kern(f2_flat, nx, ny, idx_r, coef)


def kernel(f, nodes, edges_index, W1, b1, W2, b2, Wsig, bsig, Cparam):
    f_pad = jnp.pad(f, ((0, 0), (0, 0), (0, NPAD - N)))
    f2 = _mlp_rows(f_pad, W1, b1, W2, b2)
    f2_flat = f2.reshape(B * NPAD, COUT)

    src = edges_index[..., 0]
    tgt = edges_index[..., 1]
    src_off = src + (jnp.arange(B, dtype=jnp.int32) * NPAD)[:, None]
    idx_r = jnp.stack(
        [src_off.reshape(B, NTILE, NCHUNK, CH),
         tgt.reshape(B, NTILE, NCHUNK, CH)],
        axis=3).reshape(B, NTILE, NCHUNK, 4, CH // 2)

    nx = nodes[..., 0]
    ny = nodes[..., 1]

    coef = jnp.concatenate([
        Wsig.reshape(4), bsig.reshape(2), Cparam.reshape(1),
        jnp.zeros((9,), jnp.float32)])

    out_flat = _sc_scatter(f2_flat, nx, ny, idx_r, coef)
    out = out_flat.reshape(B, ACCN, COUT)[:, :N]
    return jnp.transpose(out, (0, 2, 1))
